# baseline (speedup 1.0000x reference)
import jax
import jax.numpy as jnp
from jax.experimental import pallas as pl
from jax.experimental.pallas import tpu as pltpu

_BLOCK_S = 4096


def _pe_add_kernel(x_ref, pe_ref, o_ref):
    o_ref[...] = x_ref[...] + pe_ref[...][None, :, :]


def kernel(x, pos_emb):
    b, s, d = x.shape
    bd = 512 if d % 512 == 0 else d
    grid = (d // bd, b)
    return pl.pallas_call(
        _pe_add_kernel,
        grid=grid,
        in_specs=[
            pl.BlockSpec((1, s, bd), lambda i, j: (j, 0, i)),
            pl.BlockSpec((s, bd), lambda i, j: (0, i)),
        ],
        out_specs=pl.BlockSpec((1, s, bd), lambda i, j: (j, 0, i)),
        out_shape=jax.ShapeDtypeStruct((b, s, d), x.dtype),
        compiler_params=pltpu.CompilerParams(
            dimension_semantics=("parallel", "parallel"),
            vmem_limit_bytes=100 * 1024 * 1024,
        ),
    )(x, pos_emb)
